# baseline (speedup 1.0000x reference)
import functools

import jax
import jax.numpy as jnp
import numpy as np
from jax import lax
from jax.experimental import pallas as pl
from jax.experimental.pallas import tpu as pltpu
from jax.experimental.pallas import tpu_sc as plsc

DIM = 64
K = 2
STRIDE = 2
PAD = 0
OUT = (DIM + 2 * PAD - (K - 1) - 1) // STRIDE + 1
B = 32
C = 256
NOUT = OUT * OUT
LANE = 128
CHT = C // LANE
IC_PER_B = DIM * DIM * C // LANE
OC_PER_B = OUT * OUT * C // LANE
NW = 32

NBATCH = 64
OCB = OC_PER_B // NBATCH
GCB = OCB * K * K


def _pixel_gather_table() -> np.ndarray:
    perm_key = np.random.default_rng(1).permutation(DIM * DIM)
    new_key = np.random.default_rng(2).permutation(OUT * OUT)
    idxs = np.arange(-PAD + K // 2 - 1, DIM - PAD + K // 2 - 1, STRIDE)
    ci = np.zeros((OUT, OUT, 2), dtype=int)
    ci[:, :, 0], ci[:, :, 1] = np.meshgrid(idxs, idxs, indexing="ij")
    ip = np.transpose(ci, (2, 0, 1)).reshape(2, OUT * OUT)
    ip_s = ip[:, new_key].reshape(2, OUT, OUT)
    inv = np.empty(DIM * DIM, dtype=int)
    inv[perm_key] = np.arange(DIM * DIM)
    offs = np.array([[i, j] for i in range(K) for j in range(K)])
    rows = ip_s[0][:, :, None] + offs[None, None, :, 0]
    cols = ip_s[1][:, :, None] + offs[None, None, :, 1]
    q = rows * DIM + cols
    return inv[q].reshape(NOUT, K * K)


def _chunk_table() -> np.ndarray:
    g = _pixel_gather_table()
    tbl = np.empty((OC_PER_B, K * K), dtype=np.int32)
    for oc in range(OC_PER_B):
        i, w = divmod(oc, 64)
        jt, rem = divmod(w, 16)
        cht, js = divmod(rem, 8)
        j = jt * 8 + js
        p = i * OUT + j
        for k in range(K * K):
            r, c = divmod(int(g[p, k]), DIM)
            tbl[oc, k] = r * 128 + (c // 8) * 16 + cht * 8 + (c % 8)
    return tbl


_TBL_NP = _chunk_table()


def _pool_body(xc_hbm, tbl_hbm, out_hbm, idx_v, g_v, o_v,
               gsem0, gsem1, osem0, osem1):
    wid = lax.axis_index("s") * 2 + lax.axis_index("c")
    ibase = wid * IC_PER_B
    obase = wid * OC_PER_B
    gsems = (gsem0, gsem1)
    osems = (osem0, osem1)

    pltpu.sync_copy(tbl_hbm, idx_v)

    @plsc.parallel_loop(0, NBATCH, unroll=2)
    def _adjust(r):
        for s in range(GCB // 16):
            sl = (r, pl.ds(s * 16, 16))
            idx_v[sl] = idx_v[sl] + ibase

    def _start_gather(buf, nb):
        pltpu.async_copy(xc_hbm.at[idx_v.at[nb]], g_v.at[buf], gsems[buf])

    def _drain_gather(buf):
        pltpu.make_async_copy(
            xc_hbm.at[pl.ds(0, GCB)], g_v.at[buf], gsems[buf]).wait()

    def _drain_store(buf):
        pltpu.make_async_copy(
            o_v.at[buf], out_hbm.at[pl.ds(obase, OCB)], osems[buf]).wait()

    _start_gather(0, 0)

    @pl.loop(0, NBATCH, step=2)
    def _batch(nb):
        for h in range(2):
            cur = nb + h
            nxt = cur + 1

            @pl.when(nxt < NBATCH)
            def _():
                _start_gather(1 - h, nxt)

            _drain_gather(h)

            @pl.when(cur >= 2)
            def _():
                _drain_store(h)

            @plsc.parallel_loop(0, OCB, unroll=8)
            def _chunk(w):
                r0 = w * 4
                for s in range(LANE // 16):
                    cs = pl.ds(s * 16, 16)
                    m = jnp.maximum(
                        jnp.maximum(g_v[h, r0, cs], g_v[h, r0 + 1, cs]),
                        jnp.maximum(g_v[h, r0 + 2, cs], g_v[h, r0 + 3, cs]))
                    o_v[h, w, cs] = m

            pltpu.async_copy(
                o_v.at[h], out_hbm.at[pl.ds(obase + cur * OCB, OCB)],
                osems[h])

    _drain_store(0)
    _drain_store(1)


@functools.cache
def _pool_call():
    mesh = plsc.VectorSubcoreMesh(core_axis_name="c", subcore_axis_name="s")
    return pl.kernel(
        _pool_body,
        out_type=jax.ShapeDtypeStruct((B * OC_PER_B, LANE), jnp.float32),
        mesh=mesh,
        compiler_params=pltpu.CompilerParams(needs_layout_passes=False),
        scratch_types=[
            pltpu.VMEM((NBATCH, GCB), jnp.int32),
            pltpu.VMEM((2, GCB, LANE), jnp.float32),
            pltpu.VMEM((2, OCB, LANE), jnp.float32),
            pltpu.SemaphoreType.DMA,
            pltpu.SemaphoreType.DMA,
            pltpu.SemaphoreType.DMA,
            pltpu.SemaphoreType.DMA,
        ],
    )


def kernel(x):
    xc = (x.transpose(0, 2, 3, 1)
          .reshape(B, DIM, 8, 8, CHT, LANE)
          .transpose(0, 1, 2, 4, 3, 5)
          .reshape(B * IC_PER_B, LANE))
    res = _pool_call()(xc, jnp.asarray(_TBL_NP.reshape(NBATCH, GCB)))
    out = (res.reshape(B, OUT, 4, CHT, 8, LANE)
           .transpose(0, 1, 2, 4, 3, 5)
           .reshape(B, OUT, OUT, C)
           .transpose(0, 3, 1, 2))
    return out


# speedup vs baseline: 1.0236x; 1.0236x over previous
import functools

import jax
import jax.numpy as jnp
import numpy as np
from jax import lax
from jax.experimental import pallas as pl
from jax.experimental.pallas import tpu as pltpu
from jax.experimental.pallas import tpu_sc as plsc

DIM = 64
K = 2
STRIDE = 2
PAD = 0
OUT = (DIM + 2 * PAD - (K - 1) - 1) // STRIDE + 1
B = 32
C = 256
NOUT = OUT * OUT
LANE = 128
CHT = C // LANE
IC_PER_B = DIM * DIM * C // LANE
OC_PER_B = OUT * OUT * C // LANE
NW = 32

NBATCH = 64
OCB = OC_PER_B // NBATCH
GCB = OCB * K * K


def _pixel_gather_table() -> np.ndarray:
    perm_key = np.random.default_rng(1).permutation(DIM * DIM)
    new_key = np.random.default_rng(2).permutation(OUT * OUT)
    idxs = np.arange(-PAD + K // 2 - 1, DIM - PAD + K // 2 - 1, STRIDE)
    ci = np.zeros((OUT, OUT, 2), dtype=int)
    ci[:, :, 0], ci[:, :, 1] = np.meshgrid(idxs, idxs, indexing="ij")
    ip = np.transpose(ci, (2, 0, 1)).reshape(2, OUT * OUT)
    ip_s = ip[:, new_key].reshape(2, OUT, OUT)
    inv = np.empty(DIM * DIM, dtype=int)
    inv[perm_key] = np.arange(DIM * DIM)
    offs = np.array([[i, j] for i in range(K) for j in range(K)])
    rows = ip_s[0][:, :, None] + offs[None, None, :, 0]
    cols = ip_s[1][:, :, None] + offs[None, None, :, 1]
    q = rows * DIM + cols
    return inv[q].reshape(NOUT, K * K)


def _chunk_table() -> np.ndarray:
    g = _pixel_gather_table()
    tbl = np.empty((OC_PER_B, K * K), dtype=np.int32)
    for oc in range(OC_PER_B):
        i, w = divmod(oc, 64)
        jt, rem = divmod(w, 16)
        cht, js = divmod(rem, 8)
        j = jt * 8 + js
        p = i * OUT + j
        for k in range(K * K):
            r, c = divmod(int(g[p, k]), DIM)
            tbl[oc, k] = r * 128 + (c // 8) * 16 + cht * 8 + (c % 8)
    return tbl


_TBL_NP = _chunk_table()


def _pool_body(xc_hbm, tbl_hbm, out_hbm, idx_v, g_v, o_v,
               gsem0, gsem1, osem0, osem1):
    wid = lax.axis_index("s") * 2 + lax.axis_index("c")
    ibase = wid * IC_PER_B
    obase = wid * OC_PER_B
    gsems = (gsem0, gsem1)
    osems = (osem0, osem1)

    pltpu.sync_copy(tbl_hbm, idx_v)

    @plsc.parallel_loop(0, NBATCH, unroll=2)
    def _adjust(r):
        for s in range(GCB // 16):
            sl = (r, pl.ds(s * 16, 16))
            idx_v[sl] = idx_v[sl] + ibase

    def _start_gather(buf, nb):
        pltpu.async_copy(xc_hbm.at[idx_v.at[nb]], g_v.at[buf], gsems[buf])

    def _drain_gather(buf):
        pltpu.make_async_copy(
            xc_hbm.at[pl.ds(0, GCB)], g_v.at[buf], gsems[buf]).wait()

    def _drain_store(buf):
        pltpu.make_async_copy(
            o_v.at[buf], out_hbm.at[pl.ds(obase, OCB)], osems[buf]).wait()

    _start_gather(0, 0)

    @pl.loop(0, NBATCH, step=2)
    def _batch(nb):
        for h in range(2):
            cur = nb + h
            nxt = cur + 1

            @pl.when(nxt < NBATCH)
            def _():
                _start_gather(1 - h, nxt)

            _drain_gather(h)

            @pl.when(cur >= 2)
            def _():
                _drain_store(h)

            @plsc.parallel_loop(0, OCB, unroll=4)
            def _chunk(w):
                r0 = w * 4
                for s in range(LANE // 16):
                    cs = pl.ds(s * 16, 16)
                    m = jnp.maximum(
                        jnp.maximum(g_v[h, r0, cs], g_v[h, r0 + 1, cs]),
                        jnp.maximum(g_v[h, r0 + 2, cs], g_v[h, r0 + 3, cs]))
                    o_v[h, w, cs] = m

            pltpu.async_copy(
                o_v.at[h], out_hbm.at[pl.ds(obase + cur * OCB, OCB)],
                osems[h])

    _drain_store(0)
    _drain_store(1)


@functools.cache
def _pool_call():
    mesh = plsc.VectorSubcoreMesh(core_axis_name="c", subcore_axis_name="s")
    return pl.kernel(
        _pool_body,
        out_type=jax.ShapeDtypeStruct((B * OC_PER_B, LANE), jnp.float32),
        mesh=mesh,
        compiler_params=pltpu.CompilerParams(needs_layout_passes=False),
        scratch_types=[
            pltpu.VMEM((NBATCH, GCB), jnp.int32),
            pltpu.VMEM((2, GCB, LANE), jnp.float32),
            pltpu.VMEM((2, OCB, LANE), jnp.float32),
            pltpu.SemaphoreType.DMA,
            pltpu.SemaphoreType.DMA,
            pltpu.SemaphoreType.DMA,
            pltpu.SemaphoreType.DMA,
        ],
    )


def kernel(x):
    xc = (x.transpose(0, 2, 3, 1)
          .reshape(B, DIM, 8, 8, CHT, LANE)
          .transpose(0, 1, 2, 4, 3, 5)
          .reshape(B * IC_PER_B, LANE))
    res = _pool_call()(xc, jnp.asarray(_TBL_NP.reshape(NBATCH, GCB)))
    out = (res.reshape(B, OUT, 4, CHT, 8, LANE)
           .transpose(0, 1, 2, 4, 3, 5)
           .reshape(B, OUT, OUT, C)
           .transpose(0, 3, 1, 2))
    return out


# speedup vs baseline: 1.1902x; 1.1628x over previous
import functools

import jax
import jax.numpy as jnp
import numpy as np
from jax import lax
from jax.experimental import pallas as pl
from jax.experimental.pallas import tpu as pltpu
from jax.experimental.pallas import tpu_sc as plsc

DIM = 64
K = 2
STRIDE = 2
PAD = 0
OUT = (DIM + 2 * PAD - (K - 1) - 1) // STRIDE + 1
B = 32
C = 256
NOUT = OUT * OUT
LANE = 128
CHT = C // LANE
IC_PER_B = DIM * DIM * C // LANE
OC_PER_B = OUT * OUT * C // LANE
NW = 32

NBATCH = 64
OCB = OC_PER_B // NBATCH
GCB = OCB * K * K


def _pixel_gather_table() -> np.ndarray:
    perm_key = np.random.default_rng(1).permutation(DIM * DIM)
    new_key = np.random.default_rng(2).permutation(OUT * OUT)
    idxs = np.arange(-PAD + K // 2 - 1, DIM - PAD + K // 2 - 1, STRIDE)
    ci = np.zeros((OUT, OUT, 2), dtype=int)
    ci[:, :, 0], ci[:, :, 1] = np.meshgrid(idxs, idxs, indexing="ij")
    ip = np.transpose(ci, (2, 0, 1)).reshape(2, OUT * OUT)
    ip_s = ip[:, new_key].reshape(2, OUT, OUT)
    inv = np.empty(DIM * DIM, dtype=int)
    inv[perm_key] = np.arange(DIM * DIM)
    offs = np.array([[i, j] for i in range(K) for j in range(K)])
    rows = ip_s[0][:, :, None] + offs[None, None, :, 0]
    cols = ip_s[1][:, :, None] + offs[None, None, :, 1]
    q = rows * DIM + cols
    return inv[q].reshape(NOUT, K * K)


def _chunk_table() -> np.ndarray:
    g = _pixel_gather_table()
    tbl = np.empty((OC_PER_B, K * K), dtype=np.int32)
    for oc in range(OC_PER_B):
        i, w = divmod(oc, 64)
        jt, rem = divmod(w, 16)
        cht, js = divmod(rem, 8)
        j = jt * 8 + js
        p = i * OUT + j
        for k in range(K * K):
            r, c = divmod(int(g[p, k]), DIM)
            tbl[oc, k] = r * 128 + (c // 8) * 16 + cht * 8 + (c % 8)
    return tbl


_TBL_NP = _chunk_table()


NBUF = 4


def _pool_body(xc_hbm, tbl_hbm, out_hbm, idx_v, g_v, o_v,
               gsem0, gsem1, gsem2, gsem3, osem0, osem1, osem2, osem3):
    wid = lax.axis_index("s") * 2 + lax.axis_index("c")
    ibase = wid * IC_PER_B
    obase = wid * OC_PER_B
    gsems = (gsem0, gsem1, gsem2, gsem3)
    osems = (osem0, osem1, osem2, osem3)

    pltpu.sync_copy(tbl_hbm, idx_v)

    @plsc.parallel_loop(0, NBATCH, unroll=2)
    def _adjust(r):
        for s in range(GCB // 16):
            sl = (r, pl.ds(s * 16, 16))
            idx_v[sl] = idx_v[sl] + ibase

    def _start_gather(buf, nb):
        pltpu.async_copy(xc_hbm.at[idx_v.at[nb]], g_v.at[buf], gsems[buf])

    def _drain_gather(buf):
        pltpu.make_async_copy(
            xc_hbm.at[pl.ds(0, GCB)], g_v.at[buf], gsems[buf]).wait()

    def _drain_store(buf):
        pltpu.make_async_copy(
            o_v.at[buf], out_hbm.at[pl.ds(obase, OCB)], osems[buf]).wait()

    for i in range(NBUF - 1):
        _start_gather(i, i)

    @pl.loop(0, NBATCH, step=NBUF)
    def _batch(nb):
        for h in range(NBUF):
            cur = nb + h
            nxt = cur + NBUF - 1

            @pl.when(nxt < NBATCH)
            def _():
                _start_gather((h + NBUF - 1) % NBUF, nxt)

            _drain_gather(h)

            @pl.when(cur >= NBUF)
            def _():
                _drain_store(h)

            @plsc.parallel_loop(0, OCB, unroll=4)
            def _chunk(w):
                r0 = w * 4
                for s in range(LANE // 16):
                    cs = pl.ds(s * 16, 16)
                    m = jnp.maximum(
                        jnp.maximum(g_v[h, r0, cs], g_v[h, r0 + 1, cs]),
                        jnp.maximum(g_v[h, r0 + 2, cs], g_v[h, r0 + 3, cs]))
                    o_v[h, w, cs] = m

            pltpu.async_copy(
                o_v.at[h], out_hbm.at[pl.ds(obase + cur * OCB, OCB)],
                osems[h])

    for i in range(NBUF):
        _drain_store(i)


@functools.cache
def _pool_call():
    mesh = plsc.VectorSubcoreMesh(core_axis_name="c", subcore_axis_name="s")
    return pl.kernel(
        _pool_body,
        out_type=jax.ShapeDtypeStruct((B * OC_PER_B, LANE), jnp.float32),
        mesh=mesh,
        compiler_params=pltpu.CompilerParams(needs_layout_passes=False),
        scratch_types=[
            pltpu.VMEM((NBATCH, GCB), jnp.int32),
            pltpu.VMEM((NBUF, GCB, LANE), jnp.float32),
            pltpu.VMEM((NBUF, OCB, LANE), jnp.float32),
        ] + [pltpu.SemaphoreType.DMA] * (2 * NBUF),
    )


def kernel(x):
    xc = (x.transpose(0, 2, 3, 1)
          .reshape(B, DIM, 8, 8, CHT, LANE)
          .transpose(0, 1, 2, 4, 3, 5)
          .reshape(B * IC_PER_B, LANE))
    res = _pool_call()(xc, jnp.asarray(_TBL_NP.reshape(NBATCH, GCB)))
    out = (res.reshape(B, OUT, 4, CHT, 8, LANE)
           .transpose(0, 1, 2, 4, 3, 5)
           .reshape(B, OUT, OUT, C)
           .transpose(0, 3, 1, 2))
    return out
